# baseline (speedup 1.0000x reference)
import jax
import jax.numpy as jnp
from jax import lax
from jax.experimental import pallas as pl
from jax.experimental.pallas import tpu as pltpu
from jax.experimental.pallas import tpu_sc as plsc

N = 10000
E = 160000
D = 256
H = 256
O = 256
EPS = 1e-5

NTILES = 32
C2 = 128
C3 = 80
EP = 163840
EPT2 = EP // NTILES
T2 = EPT2 // C2
EPT3 = EP // 16
T3 = EPT3 // C3
NP = 10112
RPT = NP // 16
AW = 144
ROWB = 2000
_ECHUNKS = (64, 64, 64, 64, 64, 64, 64, 64, 64, 56)


def _stats_body(feat_ref, out_ref):
    i = pl.program_id(0)

    @pl.when(i == 0)
    def _():
        out_ref[...] = jnp.zeros_like(out_ref)

    f = feat_ref[...]
    s1 = jnp.sum(f, axis=0, keepdims=True)
    s2 = jnp.sum(f * f, axis=0, keepdims=True)
    out_ref[...] += jnp.concatenate([s1, s2], axis=0)


_stats_call = pl.pallas_call(
    _stats_body,
    grid=(N // ROWB,),
    in_specs=[pl.BlockSpec((ROWB, D), lambda i: (i, 0))],
    out_specs=pl.BlockSpec((2, D), lambda i: (0, 0)),
    out_shape=jax.ShapeDtypeStruct((2, D), jnp.float32),
)


def _qkv_body(feat_ref, sums_ref, gamma_ref, beta_ref, w_ref, b_ref,
              q_ref, k_ref, va_ref, vb_ref):
    mean = sums_ref[0:1, :] / N
    var = sums_ref[1:2, :] / N - mean * mean
    scale = gamma_ref[...] * lax.rsqrt(var + EPS)
    x = (feat_ref[...] - mean) * scale + beta_ref[...]
    y = lax.dot_general(x, w_ref[...], (((1,), (1,)), ((), ())),
                        preferred_element_type=jnp.float32) + b_ref[...]
    q_ref[...] = y[:, 0:H].astype(jnp.bfloat16)
    k_ref[...] = y[:, H:2 * H].astype(jnp.bfloat16)
    va_ref[...] = y[:, 2 * H:2 * H + 128].astype(jnp.bfloat16)
    vb_ref[...] = y[:, 2 * H + 128:2 * H + 256].astype(jnp.bfloat16)


_qkv_call = pl.pallas_call(
    _qkv_body,
    grid=(N // ROWB,),
    in_specs=[
        pl.BlockSpec((ROWB, D), lambda i: (i, 0)),
        pl.BlockSpec((2, D), lambda i: (0, 0)),
        pl.BlockSpec((1, D), lambda i: (0, 0)),
        pl.BlockSpec((1, D), lambda i: (0, 0)),
        pl.BlockSpec((2 * H + O, D), lambda i: (0, 0)),
        pl.BlockSpec((1, 2 * H + O), lambda i: (0, 0)),
    ],
    out_specs=[
        pl.BlockSpec((ROWB, H), lambda i: (i, 0)),
        pl.BlockSpec((ROWB, H), lambda i: (i, 0)),
        pl.BlockSpec((ROWB, 128), lambda i: (i, 0)),
        pl.BlockSpec((ROWB, 128), lambda i: (i, 0)),
    ],
    out_shape=[
        jax.ShapeDtypeStruct((N, H), jnp.bfloat16),
        jax.ShapeDtypeStruct((N, H), jnp.bfloat16),
        jax.ShapeDtypeStruct((N, 128), jnp.bfloat16),
        jax.ShapeDtypeStruct((N, 128), jnp.bfloat16),
    ],
)


_SC_PARAMS = pltpu.CompilerParams(needs_layout_passes=False,
                                  use_tc_tiling_on_sc=False)


def _logits_body(q_hbm, k_hbm, we_hbm, eik_hbm, w_hbm,
                 ib0, ib1, q0, k0, q1, k1, evals, wevec, tbuf,
                 semg0, semg1, semi0, semi1):
    cid = lax.axis_index("c")
    sid = lax.axis_index("s")
    wid = sid * 2 + cid
    tile_base = wid * EPT2
    pltpu.sync_copy(we_hbm, wevec)
    colidx = lax.iota(jnp.int32, 16) * 16

    def compute_chunk(qb, kb, t):
        for g in range(C2 // 16):
            def oiter(o, accs):
                wv32 = wevec[pl.ds(o * 32, 32)]
                out = []
                for l in range(16):
                    i = g * 16 + l
                    z32 = qb[i, pl.ds(o * 32, 32)] + kb[i, pl.ds(o * 32, 32)]
                    sg = 1.0 / (1.0 + jnp.exp(-z32))
                    p32 = sg * wv32
                    pe, po = plsc.unpack(p32,
                                         format=plsc.PackFormat.INTERLEAVED)
                    out.append(accs[l] + pe + po)
                return tuple(out)

            accs = lax.fori_loop(
                0, H // 32, oiter,
                tuple(jnp.zeros((16,), jnp.float32) for _ in range(16)))
            for l in range(16):
                tbuf[pl.ds(l * 16, 16)] = accs[l]
            esum = plsc.load_gather(tbuf, [colidx])
            for m in range(1, 16):
                esum = esum + plsc.load_gather(tbuf, [colidx + m])
            evals[pl.ds(t * C2 + g * 16, 16)] = jnp.exp(esum)

    def fetch_idx(ib, semi, t):
        pltpu.async_copy(eik_hbm.at[:, pl.ds(tile_base + t * C2, C2)],
                         ib, semi)

    def drain_idx(ib, semi, t):
        pltpu.make_async_copy(eik_hbm.at[:, pl.ds(tile_base + t * C2, C2)],
                              ib, semi).wait()

    def fire_rows(ib, qb, kb, semg):
        pltpu.async_copy(q_hbm.at[ib.at[0]], qb, semg)
        pltpu.async_copy(k_hbm.at[ib.at[1]], kb, semg)

    def drain_rows(ib, qb, kb, semg):
        pltpu.make_async_copy(q_hbm.at[ib.at[0]], qb, semg).wait()
        pltpu.make_async_copy(k_hbm.at[ib.at[1]], kb, semg).wait()

    fetch_idx(ib0, semi0, 0)
    drain_idx(ib0, semi0, 0)
    fire_rows(ib0, q0, k0, semg0)
    fetch_idx(ib1, semi1, 1)

    def body(u, carry):
        a = 2 * u
        drain_idx(ib1, semi1, a + 1)
        fire_rows(ib1, q1, k1, semg1)
        drain_rows(ib0, q0, k0, semg0)

        @pl.when(a + 2 < T2)
        def _():
            fetch_idx(ib0, semi0, a + 2)

        compute_chunk(q0, k0, a)

        @pl.when(a + 2 < T2)
        def _():
            drain_idx(ib0, semi0, a + 2)
            fire_rows(ib0, q0, k0, semg0)

        drain_rows(ib1, q1, k1, semg1)

        @pl.when(a + 3 < T2)
        def _():
            fetch_idx(ib1, semi1, a + 3)

        compute_chunk(q1, k1, a + 1)
        return carry

    lax.fori_loop(0, T2 // 2, body, 0)
    pltpu.sync_copy(evals, w_hbm.at[pl.ds(tile_base, EPT2)])


_logits_call = pl.kernel(
    _logits_body,
    out_type=jax.ShapeDtypeStruct((EP,), jnp.float32),
    mesh=plsc.VectorSubcoreMesh(core_axis_name="c", subcore_axis_name="s"),
    compiler_params=_SC_PARAMS,
    scratch_types=[
        pltpu.VMEM((2, C2), jnp.int32),
        pltpu.VMEM((2, C2), jnp.int32),
        pltpu.VMEM((C2, H), jnp.bfloat16),
        pltpu.VMEM((C2, H), jnp.bfloat16),
        pltpu.VMEM((C2, H), jnp.bfloat16),
        pltpu.VMEM((C2, H), jnp.bfloat16),
        pltpu.VMEM((EPT2,), jnp.float32),
        pltpu.VMEM((H,), jnp.bfloat16),
        pltpu.VMEM((H,), jnp.float32),
        pltpu.SemaphoreType.DMA,
        pltpu.SemaphoreType.DMA,
        pltpu.SemaphoreType.DMA,
        pltpu.SemaphoreType.DMA,
    ],
)


def _agg_body(va_hbm, vb_hbm, ei3_hbm, w_hbm, out_hbm,
              acc_v, acc_d, ib0, ib1, w0, w1, didx0, didx1, v0, v1,
              stg0, stg1, ws0, ws1, semv0, semv1, semi0, semi1,
              semsc0, semsc1):
    cid = lax.axis_index("c")
    sid = lax.axis_index("s")
    tile_base = sid * EPT3

    def zrow(i, c):
        for o in range(8):
            stg0[i, pl.ds(o * 16, 16)] = jnp.zeros((16,), jnp.float32)
        ws0[i, pl.ds(0, 16)] = jnp.zeros((16,), jnp.float32)
        return c

    lax.fori_loop(0, 64, zrow, 0)
    row0 = sid * RPT
    off = 0
    for cnt in _ECHUNKS:
        pltpu.sync_copy(stg0.at[pl.ds(0, cnt)],
                        acc_v.at[pl.ds(row0 + off, cnt)])
        pltpu.sync_copy(ws0.at[pl.ds(0, cnt)],
                        acc_d.at[pl.ds(row0 + off, cnt)])
        off += cnt
    plsc.subcore_barrier()

    lane0 = (lax.iota(jnp.int32, 16) == 0).astype(jnp.float32)

    def fetch_idx(ib, wb, semi, t):
        base = tile_base + t * C3
        pltpu.async_copy(ei3_hbm.at[:, pl.ds(base, C3)], ib, semi)
        pltpu.async_copy(w_hbm.at[pl.ds(base, C3)], wb, semi)

    def drain_idx(ib, wb, semi, t):
        base = tile_base + t * C3
        pltpu.make_async_copy(ei3_hbm.at[:, pl.ds(base, C3)], ib, semi).wait()
        pltpu.make_async_copy(w_hbm.at[pl.ds(base, C3)], wb, semi).wait()

    def fire_rows(ib, vb, semv):
        @pl.when(cid == 0)
        def _():
            pltpu.async_copy(va_hbm.at[ib.at[0]], vb, semv)

        @pl.when(cid == 1)
        def _():
            pltpu.async_copy(vb_hbm.at[ib.at[0]], vb, semv)

    def drain_rows(ib, vb, semv):
        pltpu.make_async_copy(va_hbm.at[ib.at[0]], vb, semv).wait()

    def build(ib, wb, vb, stg, wstg, didx):
        for g in range(C3 // 16):
            wvec = wb[pl.ds(g * 16, 16)]
            for l in range(16):
                i = g * 16 + l
                w = wvec[l]
                for o in range(4):
                    v32 = vb[i, pl.ds(o * 32, 32)]
                    ve, vo = plsc.unpack(v32,
                                         format=plsc.PackFormat.INTERLEAVED)
                    stg[i, pl.ds(o * 32, 16)] = ve * w
                    stg[i, pl.ds(o * 32 + 16, 16)] = vo * w
                wstg[i, pl.ds(0, 16)] = lane0 * w
        for o in range(C3 // 16):
            didx[pl.ds(o * 16, 16)] = ib[1, pl.ds(o * 16, 16)]

    def fire_scatter(vb, wstg, didx, semsc):
        pltpu.async_copy(vb, acc_v.at[didx], semsc, add=True)
        pltpu.async_copy(wstg, acc_d.at[didx], semsc, add=True)

    def drain_scatter(vb, wstg, didx, semsc):
        pltpu.make_async_copy(vb, acc_v.at[didx], semsc).wait()
        pltpu.make_async_copy(wstg, acc_d.at[didx], semsc).wait()

    fetch_idx(ib0, w0, semi0, 0)
    drain_idx(ib0, w0, semi0, 0)
    fire_rows(ib0, v0, semv0)
    fetch_idx(ib1, w1, semi1, 1)

    def body(u, carry):
        a = 2 * u

        drain_idx(ib1, w1, semi1, a + 1)

        @pl.when(u > 0)
        def _():
            drain_scatter(stg1, ws1, didx1, semsc1)

        fire_rows(ib1, v1, semv1)
        drain_rows(ib0, v0, semv0)
        build(ib0, w0, v0, stg0, ws0, didx0)
        fire_scatter(stg0, ws0, didx0, semsc0)

        @pl.when(a + 2 < T3)
        def _():
            fetch_idx(ib0, w0, semi0, a + 2)

        drain_rows(ib1, v1, semv1)
        build(ib1, w1, v1, stg1, ws1, didx1)
        fire_scatter(stg1, ws1, didx1, semsc1)

        @pl.when(a + 2 < T3)
        def _():
            drain_idx(ib0, w0, semi0, a + 2)
            drain_scatter(stg0, ws0, didx0, semsc0)
            fire_rows(ib0, v0, semv0)

        @pl.when(a + 3 < T3)
        def _():
            fetch_idx(ib1, w1, semi1, a + 3)

        return carry

    lax.fori_loop(0, T3 // 2, body, 0)

    drain_scatter(stg0, ws0, didx0, semsc0)
    drain_scatter(stg1, ws1, didx1, semsc1)
    plsc.subcore_barrier()

    j16 = lax.iota(jnp.int32, 16)
    colperm = []
    for m in range(8):
        cc = m * 16 + j16
        colperm.append((cc // 32) * 32 + (cc % 2) * 16 + (cc % 32) // 2)

    off = 0
    for cnt in _ECHUNKS:
        pltpu.sync_copy(acc_v.at[pl.ds(row0 + off, cnt)],
                        stg0.at[pl.ds(0, cnt)])
        pltpu.sync_copy(acc_d.at[pl.ds(row0 + off, cnt)],
                        ws0.at[pl.ds(0, cnt)])

        def rowfn(i, c):
            dvec = ws0[i, pl.ds(0, 16)]
            ivec = jnp.where(dvec != 0.0, 1.0 / dvec, 0.0)
            inv = ivec[0]
            rows16 = jnp.full((16,), i, jnp.int32)
            gs = [plsc.load_gather(stg0, [rows16, colperm[m]])
                  for m in range(8)]
            for m in range(8):
                stg0[i, pl.ds(m * 16, 16)] = gs[m] * inv
            return c

        lax.fori_loop(0, cnt, rowfn, 0)

        @pl.when(cid == 0)
        def _():
            pltpu.sync_copy(stg0.at[pl.ds(0, cnt)],
                            out_hbm.at[pl.ds(row0 + off, cnt), pl.ds(0, 128)])

        @pl.when(cid == 1)
        def _():
            pltpu.sync_copy(stg0.at[pl.ds(0, cnt)],
                            out_hbm.at[pl.ds(row0 + off, cnt),
                                       pl.ds(128, 128)])

        off += cnt


_agg_call = pl.kernel(
    _agg_body,
    out_type=jax.ShapeDtypeStruct((NP, 2 * 128), jnp.float32),
    mesh=plsc.VectorSubcoreMesh(core_axis_name="c", subcore_axis_name="s"),
    compiler_params=_SC_PARAMS,
    scratch_types=[
        pltpu.VMEM_SHARED((NP, 128), jnp.float32),
        pltpu.VMEM_SHARED((NP, 16), jnp.float32),
        pltpu.VMEM((2, C3), jnp.int32),
        pltpu.VMEM((2, C3), jnp.int32),
        pltpu.VMEM((C3,), jnp.float32),
        pltpu.VMEM((C3,), jnp.float32),
        pltpu.VMEM((C3,), jnp.int32),
        pltpu.VMEM((C3,), jnp.int32),
        pltpu.VMEM((C3, 128), jnp.bfloat16),
        pltpu.VMEM((C3, 128), jnp.bfloat16),
        pltpu.VMEM((C3, 128), jnp.float32),
        pltpu.VMEM((C3, 128), jnp.float32),
        pltpu.VMEM((C3, 16), jnp.float32),
        pltpu.VMEM((C3, 16), jnp.float32),
        pltpu.SemaphoreType.DMA,
        pltpu.SemaphoreType.DMA,
        pltpu.SemaphoreType.DMA,
        pltpu.SemaphoreType.DMA,
        pltpu.SemaphoreType.DMA,
        pltpu.SemaphoreType.DMA,
    ],
)


@jax.jit
def kernel(feat, edge_index, bn_gamma, bn_beta, Wq, bq, Wk, Wv, We):
    src = edge_index[0]
    dst = edge_index[1]
    pad = EP - E
    srcp = jnp.concatenate([src, jnp.zeros((pad,), jnp.int32)])
    dstk = jnp.concatenate([dst, jnp.zeros((pad,), jnp.int32)])
    dsts = jnp.concatenate([dst, jnp.full((pad,), N, jnp.int32)])
    eik = jnp.stack([srcp, dstk])
    ei3 = jnp.stack([srcp, dsts])
    W = jnp.concatenate([Wq, Wk, Wv], axis=0)
    b = jnp.concatenate([bq, jnp.zeros((2 * H,), jnp.float32)]).reshape(1, -1)
    gamma = bn_gamma.reshape(1, D)
    beta = bn_beta.reshape(1, D)
    we = We.reshape(H).astype(jnp.bfloat16)

    sums = _stats_call(feat)
    q, k, va, vb = _qkv_call(feat, sums, gamma, beta, W, b)
    w = _logits_call(q, k, we, eik)
    out = _agg_call(va, vb, ei3, w)
    return out[:N]


# speedup vs baseline: 1.0250x; 1.0250x over previous
import jax
import jax.numpy as jnp
from jax import lax
from jax.experimental import pallas as pl
from jax.experimental.pallas import tpu as pltpu
from jax.experimental.pallas import tpu_sc as plsc

N = 10000
E = 160000
D = 256
H = 256
O = 256
EPS = 1e-5

NTILES = 32
C2 = 128
C3 = 64
EP = 163840
EPT2 = EP // NTILES
T2 = EPT2 // C2
EPT3 = EP // 16
T3 = EPT3 // C3
NP = 10112
RPT = NP // 16
AW = 144
ROWB = 2000
_ECHUNKS = (64, 64, 64, 64, 64, 64, 64, 64, 64, 56)


def _stats_body(feat_ref, out_ref):
    i = pl.program_id(0)

    @pl.when(i == 0)
    def _():
        out_ref[...] = jnp.zeros_like(out_ref)

    f = feat_ref[...]
    s1 = jnp.sum(f, axis=0, keepdims=True)
    s2 = jnp.sum(f * f, axis=0, keepdims=True)
    out_ref[...] += jnp.concatenate([s1, s2], axis=0)


_stats_call = pl.pallas_call(
    _stats_body,
    grid=(N // ROWB,),
    in_specs=[pl.BlockSpec((ROWB, D), lambda i: (i, 0))],
    out_specs=pl.BlockSpec((2, D), lambda i: (0, 0)),
    out_shape=jax.ShapeDtypeStruct((2, D), jnp.float32),
)


def _qkv_body(feat_ref, sums_ref, gamma_ref, beta_ref, w_ref, b_ref,
              q_ref, k_ref, va_ref, vb_ref):
    mean = sums_ref[0:1, :] / N
    var = sums_ref[1:2, :] / N - mean * mean
    scale = gamma_ref[...] * lax.rsqrt(var + EPS)
    x = (feat_ref[...] - mean) * scale + beta_ref[...]
    y = lax.dot_general(x, w_ref[...], (((1,), (1,)), ((), ())),
                        preferred_element_type=jnp.float32) + b_ref[...]
    q_ref[...] = y[:, 0:H].astype(jnp.bfloat16)
    k_ref[...] = y[:, H:2 * H].astype(jnp.bfloat16)
    va_ref[...] = y[:, 2 * H:2 * H + 128].astype(jnp.bfloat16)
    vb_ref[...] = y[:, 2 * H + 128:2 * H + 256].astype(jnp.bfloat16)


_qkv_call = pl.pallas_call(
    _qkv_body,
    grid=(N // ROWB,),
    in_specs=[
        pl.BlockSpec((ROWB, D), lambda i: (i, 0)),
        pl.BlockSpec((2, D), lambda i: (0, 0)),
        pl.BlockSpec((1, D), lambda i: (0, 0)),
        pl.BlockSpec((1, D), lambda i: (0, 0)),
        pl.BlockSpec((2 * H + O, D), lambda i: (0, 0)),
        pl.BlockSpec((1, 2 * H + O), lambda i: (0, 0)),
    ],
    out_specs=[
        pl.BlockSpec((ROWB, H), lambda i: (i, 0)),
        pl.BlockSpec((ROWB, H), lambda i: (i, 0)),
        pl.BlockSpec((ROWB, 128), lambda i: (i, 0)),
        pl.BlockSpec((ROWB, 128), lambda i: (i, 0)),
    ],
    out_shape=[
        jax.ShapeDtypeStruct((N, H), jnp.bfloat16),
        jax.ShapeDtypeStruct((N, H), jnp.bfloat16),
        jax.ShapeDtypeStruct((N, 128), jnp.bfloat16),
        jax.ShapeDtypeStruct((N, 128), jnp.bfloat16),
    ],
)


_SC_PARAMS = pltpu.CompilerParams(needs_layout_passes=False,
                                  use_tc_tiling_on_sc=False)


def _logits_body(q_hbm, k_hbm, we_hbm, eik_hbm, w_hbm,
                 ib0, ib1, q0, k0, q1, k1, evals, wevec, tbuf,
                 semg0, semg1, semi0, semi1):
    cid = lax.axis_index("c")
    sid = lax.axis_index("s")
    wid = sid * 2 + cid
    tile_base = wid * EPT2
    pltpu.sync_copy(we_hbm, wevec)
    colidx = lax.iota(jnp.int32, 16) * 16

    def compute_chunk(qb, kb, t):
        for g in range(C2 // 16):
            def oiter(o, accs):
                wv32 = wevec[pl.ds(o * 32, 32)]
                out = []
                for l in range(16):
                    i = g * 16 + l
                    z32 = qb[i, pl.ds(o * 32, 32)] + kb[i, pl.ds(o * 32, 32)]
                    sg = 1.0 / (1.0 + jnp.exp(-z32))
                    p32 = sg * wv32
                    pe, po = plsc.unpack(p32,
                                         format=plsc.PackFormat.INTERLEAVED)
                    out.append(accs[l] + pe + po)
                return tuple(out)

            accs = lax.fori_loop(
                0, H // 32, oiter,
                tuple(jnp.zeros((16,), jnp.float32) for _ in range(16)))
            for l in range(16):
                tbuf[pl.ds(l * 16, 16)] = accs[l]
            esum = plsc.load_gather(tbuf, [colidx])
            for m in range(1, 16):
                esum = esum + plsc.load_gather(tbuf, [colidx + m])
            evals[pl.ds(t * C2 + g * 16, 16)] = jnp.exp(esum)

    def fetch_idx(ib, semi, t):
        pltpu.async_copy(eik_hbm.at[:, pl.ds(tile_base + t * C2, C2)],
                         ib, semi)

    def drain_idx(ib, semi, t):
        pltpu.make_async_copy(eik_hbm.at[:, pl.ds(tile_base + t * C2, C2)],
                              ib, semi).wait()

    def fire_rows(ib, qb, kb, semg):
        pltpu.async_copy(q_hbm.at[ib.at[0]], qb, semg)
        pltpu.async_copy(k_hbm.at[ib.at[1]], kb, semg)

    def drain_rows(ib, qb, kb, semg):
        pltpu.make_async_copy(q_hbm.at[ib.at[0]], qb, semg).wait()
        pltpu.make_async_copy(k_hbm.at[ib.at[1]], kb, semg).wait()

    fetch_idx(ib0, semi0, 0)
    drain_idx(ib0, semi0, 0)
    fire_rows(ib0, q0, k0, semg0)
    fetch_idx(ib1, semi1, 1)

    def body(u, carry):
        a = 2 * u
        drain_idx(ib1, semi1, a + 1)
        fire_rows(ib1, q1, k1, semg1)
        drain_rows(ib0, q0, k0, semg0)

        @pl.when(a + 2 < T2)
        def _():
            fetch_idx(ib0, semi0, a + 2)

        compute_chunk(q0, k0, a)

        @pl.when(a + 2 < T2)
        def _():
            drain_idx(ib0, semi0, a + 2)
            fire_rows(ib0, q0, k0, semg0)

        drain_rows(ib1, q1, k1, semg1)

        @pl.when(a + 3 < T2)
        def _():
            fetch_idx(ib1, semi1, a + 3)

        compute_chunk(q1, k1, a + 1)
        return carry

    lax.fori_loop(0, T2 // 2, body, 0)
    pltpu.sync_copy(evals, w_hbm.at[pl.ds(tile_base, EPT2)])


_logits_call = pl.kernel(
    _logits_body,
    out_type=jax.ShapeDtypeStruct((EP,), jnp.float32),
    mesh=plsc.VectorSubcoreMesh(core_axis_name="c", subcore_axis_name="s"),
    compiler_params=_SC_PARAMS,
    scratch_types=[
        pltpu.VMEM((2, C2), jnp.int32),
        pltpu.VMEM((2, C2), jnp.int32),
        pltpu.VMEM((C2, H), jnp.bfloat16),
        pltpu.VMEM((C2, H), jnp.bfloat16),
        pltpu.VMEM((C2, H), jnp.bfloat16),
        pltpu.VMEM((C2, H), jnp.bfloat16),
        pltpu.VMEM((EPT2,), jnp.float32),
        pltpu.VMEM((H,), jnp.bfloat16),
        pltpu.VMEM((H,), jnp.float32),
        pltpu.SemaphoreType.DMA,
        pltpu.SemaphoreType.DMA,
        pltpu.SemaphoreType.DMA,
        pltpu.SemaphoreType.DMA,
    ],
)


def _agg_body(va_hbm, vb_hbm, ei3_hbm, w_hbm, out_hbm,
              acc_v, acc_d, ib0, ib1, w0, w1, didx0, didx1, v0, v1,
              stg0, stg1, ws0, ws1, semv0, semv1, semi0, semi1,
              semsc0, semsc1):
    cid = lax.axis_index("c")
    sid = lax.axis_index("s")
    tile_base = sid * EPT3

    def zrow(i, c):
        for o in range(8):
            stg0[i, pl.ds(o * 16, 16)] = jnp.zeros((16,), jnp.float32)
        ws0[i, pl.ds(0, 16)] = jnp.zeros((16,), jnp.float32)
        return c

    lax.fori_loop(0, 64, zrow, 0)
    row0 = sid * RPT
    off = 0
    for cnt in _ECHUNKS:
        pltpu.sync_copy(stg0.at[pl.ds(0, cnt)],
                        acc_v.at[pl.ds(row0 + off, cnt)])
        pltpu.sync_copy(ws0.at[pl.ds(0, cnt)],
                        acc_d.at[pl.ds(row0 + off, cnt)])
        off += cnt
    plsc.subcore_barrier()

    lane0 = (lax.iota(jnp.int32, 16) == 0).astype(jnp.float32)

    def fetch_idx(ib, wb, semi, t):
        base = tile_base + t * C3
        pltpu.async_copy(ei3_hbm.at[:, pl.ds(base, C3)], ib, semi)
        pltpu.async_copy(w_hbm.at[pl.ds(base, C3)], wb, semi)

    def drain_idx(ib, wb, semi, t):
        base = tile_base + t * C3
        pltpu.make_async_copy(ei3_hbm.at[:, pl.ds(base, C3)], ib, semi).wait()
        pltpu.make_async_copy(w_hbm.at[pl.ds(base, C3)], wb, semi).wait()

    def fire_rows(ib, vb, semv):
        @pl.when(cid == 0)
        def _():
            pltpu.async_copy(va_hbm.at[ib.at[0]], vb, semv)

        @pl.when(cid == 1)
        def _():
            pltpu.async_copy(vb_hbm.at[ib.at[0]], vb, semv)

    def drain_rows(ib, vb, semv):
        pltpu.make_async_copy(va_hbm.at[ib.at[0]], vb, semv).wait()

    def build(ib, wb, vb, stg, wstg, didx):
        for g in range(C3 // 16):
            wvec = wb[pl.ds(g * 16, 16)]
            for l in range(16):
                i = g * 16 + l
                w = wvec[l]
                for o in range(4):
                    v32 = vb[i, pl.ds(o * 32, 32)]
                    ve, vo = plsc.unpack(v32,
                                         format=plsc.PackFormat.INTERLEAVED)
                    stg[i, pl.ds(o * 32, 16)] = ve * w
                    stg[i, pl.ds(o * 32 + 16, 16)] = vo * w
                wstg[i, pl.ds(0, 16)] = lane0 * w
        for o in range(C3 // 16):
            didx[pl.ds(o * 16, 16)] = ib[1, pl.ds(o * 16, 16)]

    def fire_scatter(vb, wstg, didx, semsc):
        pltpu.async_copy(vb, acc_v.at[didx], semsc, add=True)
        pltpu.async_copy(wstg, acc_d.at[didx], semsc, add=True)

    def drain_scatter(vb, wstg, didx, semsc):
        pltpu.make_async_copy(vb, acc_v.at[didx], semsc).wait()
        pltpu.make_async_copy(wstg, acc_d.at[didx], semsc).wait()

    fetch_idx(ib0, w0, semi0, 0)
    drain_idx(ib0, w0, semi0, 0)
    fire_rows(ib0, v0, semv0)
    fetch_idx(ib1, w1, semi1, 1)

    def body(u, carry):
        a = 2 * u

        drain_idx(ib1, w1, semi1, a + 1)

        @pl.when(u > 0)
        def _():
            drain_scatter(stg1, ws1, didx1, semsc1)

        fire_rows(ib1, v1, semv1)
        drain_rows(ib0, v0, semv0)
        build(ib0, w0, v0, stg0, ws0, didx0)
        fire_scatter(stg0, ws0, didx0, semsc0)

        @pl.when(a + 2 < T3)
        def _():
            fetch_idx(ib0, w0, semi0, a + 2)

        drain_rows(ib1, v1, semv1)
        build(ib1, w1, v1, stg1, ws1, didx1)
        fire_scatter(stg1, ws1, didx1, semsc1)

        @pl.when(a + 2 < T3)
        def _():
            drain_idx(ib0, w0, semi0, a + 2)
            drain_scatter(stg0, ws0, didx0, semsc0)
            fire_rows(ib0, v0, semv0)

        @pl.when(a + 3 < T3)
        def _():
            fetch_idx(ib1, w1, semi1, a + 3)

        return carry

    lax.fori_loop(0, T3 // 2, body, 0)

    drain_scatter(stg0, ws0, didx0, semsc0)
    drain_scatter(stg1, ws1, didx1, semsc1)
    plsc.subcore_barrier()

    j16 = lax.iota(jnp.int32, 16)
    colperm = []
    for m in range(8):
        cc = m * 16 + j16
        colperm.append((cc // 32) * 32 + (cc % 2) * 16 + (cc % 32) // 2)

    off = 0
    for cnt in _ECHUNKS:
        pltpu.sync_copy(acc_v.at[pl.ds(row0 + off, cnt)],
                        stg0.at[pl.ds(0, cnt)])
        pltpu.sync_copy(acc_d.at[pl.ds(row0 + off, cnt)],
                        ws0.at[pl.ds(0, cnt)])

        def rowfn(i, c):
            dvec = ws0[i, pl.ds(0, 16)]
            ivec = jnp.where(dvec != 0.0, 1.0 / dvec, 0.0)
            inv = ivec[0]
            rows16 = jnp.full((16,), i, jnp.int32)
            gs = [plsc.load_gather(stg0, [rows16, colperm[m]])
                  for m in range(8)]
            for m in range(8):
                stg0[i, pl.ds(m * 16, 16)] = gs[m] * inv
            return c

        lax.fori_loop(0, cnt, rowfn, 0)

        @pl.when(cid == 0)
        def _():
            pltpu.sync_copy(stg0.at[pl.ds(0, cnt)],
                            out_hbm.at[pl.ds(row0 + off, cnt), pl.ds(0, 128)])

        @pl.when(cid == 1)
        def _():
            pltpu.sync_copy(stg0.at[pl.ds(0, cnt)],
                            out_hbm.at[pl.ds(row0 + off, cnt),
                                       pl.ds(128, 128)])

        off += cnt


_agg_call = pl.kernel(
    _agg_body,
    out_type=jax.ShapeDtypeStruct((NP, 2 * 128), jnp.float32),
    mesh=plsc.VectorSubcoreMesh(core_axis_name="c", subcore_axis_name="s"),
    compiler_params=_SC_PARAMS,
    scratch_types=[
        pltpu.VMEM_SHARED((NP, 128), jnp.float32),
        pltpu.VMEM_SHARED((NP, 16), jnp.float32),
        pltpu.VMEM((2, C3), jnp.int32),
        pltpu.VMEM((2, C3), jnp.int32),
        pltpu.VMEM((C3,), jnp.float32),
        pltpu.VMEM((C3,), jnp.float32),
        pltpu.VMEM((C3,), jnp.int32),
        pltpu.VMEM((C3,), jnp.int32),
        pltpu.VMEM((C3, 128), jnp.bfloat16),
        pltpu.VMEM((C3, 128), jnp.bfloat16),
        pltpu.VMEM((C3, 128), jnp.float32),
        pltpu.VMEM((C3, 128), jnp.float32),
        pltpu.VMEM((C3, 16), jnp.float32),
        pltpu.VMEM((C3, 16), jnp.float32),
        pltpu.SemaphoreType.DMA,
        pltpu.SemaphoreType.DMA,
        pltpu.SemaphoreType.DMA,
        pltpu.SemaphoreType.DMA,
        pltpu.SemaphoreType.DMA,
        pltpu.SemaphoreType.DMA,
    ],
)


@jax.jit
def kernel(feat, edge_index, bn_gamma, bn_beta, Wq, bq, Wk, Wv, We):
    src = edge_index[0]
    dst = edge_index[1]
    pad = EP - E
    srcp = jnp.concatenate([src, jnp.zeros((pad,), jnp.int32)])
    dstk = jnp.concatenate([dst, jnp.zeros((pad,), jnp.int32)])
    dsts = jnp.concatenate([dst, jnp.full((pad,), N, jnp.int32)])
    eik = jnp.stack([srcp, dstk])
    ei3 = jnp.stack([srcp, dsts])
    W = jnp.concatenate([Wq, Wk, Wv], axis=0)
    b = jnp.concatenate([bq, jnp.zeros((2 * H,), jnp.float32)]).reshape(1, -1)
    gamma = bn_gamma.reshape(1, D)
    beta = bn_beta.reshape(1, D)
    we = We.reshape(H).astype(jnp.bfloat16)

    sums = _stats_call(feat)
    q, k, va, vb = _qkv_call(feat, sums, gamma, beta, W, b)
    w = _logits_call(q, k, we, eik)
    out = _agg_call(va, vb, ei3, w)
    return out[:N]
